# baseline (speedup 1.0000x reference)
import jax
import jax.numpy as jnp
from jax import lax
from jax.experimental import pallas as pl
from jax.experimental.pallas import tpu as pltpu
from jax.experimental.pallas import tpu_sc as plsc

VOCAB_SIZE = 1000000
EMB_SIZE = 32
N = 16384
LANES = 16

NUM_CORES = 2
NUM_SUBCORES = 16
NUM_WORKERS = NUM_CORES * NUM_SUBCORES

FULL_TILES = VOCAB_SIZE // 128
ROW_BLOCKS = FULL_TILES + 1
TILES_PER_PART = FULL_TILES // 8
EXTRA_TILES = FULL_TILES % 8
TAIL_START = FULL_TILES * 128
TAIL = VOCAB_SIZE - TAIL_START
ROW_STRIDE = ROW_BLOCKS * 128
TRASH_ROW = EMB_SIZE * ROW_BLOCKS
SCRATCH_ROWS = TRASH_ROW + 8
SLOTS = TILES_PER_PART + 2


def _scatter_rows():
    w = jnp.arange(NUM_WORKERS)
    slab, part = w // 8, w % 8
    r = jnp.arange(8)
    feat_rows = (slab[:, None] * 8 + r[None, :]) * ROW_BLOCKS
    k = jnp.arange(TILES_PER_PART)
    main = (feat_rows[:, None, :]
            + (part[:, None] * TILES_PER_PART + k[None, :])[:, :, None])
    extra = jnp.where((part < EXTRA_TILES)[:, None],
                      feat_rows + (8 * TILES_PER_PART + part)[:, None],
                      TRASH_ROW + r[None, :])
    tail = jnp.where((part == 0)[:, None],
                     feat_rows + FULL_TILES,
                     TRASH_ROW + r[None, :])
    return jnp.concatenate(
        [main, extra[:, None, :], tail[:, None, :]], axis=1
    ).astype(jnp.int32)


def _detile_body(table_t, aux, ridx, scratch, tile_v, idxbuf_v, sem):
    c = lax.axis_index("c")
    s = lax.axis_index("s")
    wid = c * NUM_SUBCORES + s
    slab = wid // 8
    part = wid % 8
    pltpu.sync_copy(ridx.at[wid], idxbuf_v)

    def step(k, carry):
        block = part * TILES_PER_PART + k
        pltpu.sync_copy(
            table_t.at[pl.ds(slab * 8, 8), pl.ds(block * 128, 128)], tile_v
        )
        pltpu.async_copy(tile_v, scratch.at[idxbuf_v.at[k]], sem).wait()
        return carry

    lax.fori_loop(0, TILES_PER_PART, step, 0)

    eblock = jnp.where(part < EXTRA_TILES, 8 * TILES_PER_PART + part, 0)
    pltpu.sync_copy(
        table_t.at[pl.ds(slab * 8, 8), pl.ds(eblock * 128, 128)], tile_v
    )
    pltpu.async_copy(
        tile_v, scratch.at[idxbuf_v.at[TILES_PER_PART]], sem
    ).wait()

    pltpu.sync_copy(aux.at[pl.ds(slab * 8, 8)], tile_v)
    pltpu.async_copy(
        tile_v, scratch.at[idxbuf_v.at[TILES_PER_PART + 1]], sem
    ).wait()


def _gather_body(flat, vocab, out_flat, idx_v, vals_v, sem):
    c = lax.axis_index("c")
    s = lax.axis_index("s")
    wid = c * NUM_SUBCORES + s
    pltpu.sync_copy(vocab, idx_v)
    row_base = wid * ROW_STRIDE

    def add_base(g, carry):
        idx_v[pl.ds(g * LANES, LANES)] = (
            idx_v[pl.ds(g * LANES, LANES)] + row_base
        )
        return carry

    lax.fori_loop(0, N // LANES, add_base, 0)
    pltpu.async_copy(flat.at[idx_v], vals_v, sem).wait()
    pltpu.sync_copy(vals_v, out_flat.at[pl.ds(wid * N, N)])


def _mesh():
    return plsc.VectorSubcoreMesh(
        core_axis_name="c", subcore_axis_name="s",
        num_cores=NUM_CORES, num_subcores=NUM_SUBCORES,
    )


def _detile(table_t, aux, ridx):
    return pl.kernel(
        _detile_body,
        out_type=jax.ShapeDtypeStruct((SCRATCH_ROWS, 128), jnp.float32),
        mesh=_mesh(),
        scratch_types=[
            pltpu.VMEM((8, 128), jnp.float32),
            pltpu.VMEM((SLOTS, 8), jnp.int32),
            pltpu.SemaphoreType.DMA,
        ],
    )(table_t, aux, ridx)


def _node_gather(flat, vocab_ids):
    return pl.kernel(
        _gather_body,
        out_type=jax.ShapeDtypeStruct((EMB_SIZE * N,), jnp.float32),
        mesh=_mesh(),
        scratch_types=[
            pltpu.VMEM((N,), jnp.int32),
            pltpu.VMEM((N,), jnp.float32),
            pltpu.SemaphoreType.DMA,
        ],
    )(flat, vocab_ids)


@jax.jit
def _impl(vocab_ids, selector_ids, node_table, sel_table):
    vidx = vocab_ids.astype(jnp.int32)
    table_t = node_table.T
    aux = jnp.pad(table_t[:, TAIL_START:], ((0, 0), (0, 128 - TAIL)))
    scratch = _detile(table_t, aux, _scatter_rows())
    out_t = _node_gather(scratch.reshape(SCRATCH_ROWS * 128), vidx)
    nodes = out_t.reshape(EMB_SIZE, N).T
    sel = jnp.take(sel_table, selector_ids.astype(jnp.int32), axis=0)
    return jnp.concatenate([nodes, sel], axis=1)


def kernel(vocab_ids, selector_ids, node_table, sel_table):
    return _impl(vocab_ids, selector_ids, node_table, sel_table)
